# baseline (speedup 1.0000x reference)
import functools

import jax
import jax.numpy as jnp
from jax import lax
from jax.experimental import pallas as pl
from jax.experimental.pallas import tpu as pltpu
from jax.experimental.pallas import tpu_sc as plsc

B, N, D = 32, 576, 768
ROWS = B * N
TC_ROWS = 12288
SC_ROWS = ROWS - TC_ROWS
BLK = 3072
NW = 32
RPW = SC_ROWS // NW
CH = 48
NCH = RPW // CH
NL = D // 16


def _tc_body(idx_ref, embed_ref, tok_ref, out_ref):
    row = embed_ref[pl.ds(idx_ref[0], 1), :]
    out_ref[...] = tok_ref[...] + row


def _sc_body(tok_hbm, embed_hbm, idx_hbm, out_hbm,
             idx_v, rows_v, buf, sem_in0, sem_in1, sem_out0, sem_out1):
    wid = lax.axis_index("s") * 2 + lax.axis_index("c")
    base = wid * RPW

    pltpu.sync_copy(idx_hbm, idx_v)
    pltpu.async_copy(embed_hbm.at[idx_v], rows_v, sem_in0).wait()
    row = [rows_v[0, pl.ds(j * 16, 16)] for j in range(NL)]

    sem_in = (sem_in0, sem_in1)
    sem_out = (sem_out0, sem_out1)

    def start_in(k):
        return pltpu.async_copy(
            tok_hbm.at[pl.ds(TC_ROWS + base + k * CH, CH)], buf.at[k % 2],
            sem_in[k % 2])

    def start_out(k):
        return pltpu.async_copy(
            buf.at[k % 2], out_hbm.at[pl.ds(base + k * CH, CH)],
            sem_out[k % 2])

    in_h = [None] * NCH
    out_h = [None] * NCH
    in_h[0] = start_in(0)
    for k in range(NCH):
        b = k % 2
        if k + 1 < NCH:
            if k >= 1:
                out_h[k - 1].wait()
            in_h[k + 1] = start_in(k + 1)
        in_h[k].wait()

        def row_body(r, carry):
            for j in range(NL):
                sl = pl.ds(j * 16, 16)
                buf[b, r, sl] = buf[b, r, sl] + row[j]
            return carry
        lax.fori_loop(0, CH, row_body, None)
        out_h[k] = start_out(k)
    out_h[NCH - 2].wait()
    out_h[NCH - 1].wait()


def kernel(tokens, embed, frame_idx):
    idx = jnp.asarray(frame_idx, dtype=jnp.int32).reshape((1,))
    idx8 = jnp.full((8,), frame_idx, dtype=jnp.int32)
    tok2 = tokens.reshape(ROWS, D)

    tc_out = pl.pallas_call(
        _tc_body,
        grid=(TC_ROWS // BLK,),
        in_specs=[
            pl.BlockSpec(memory_space=pltpu.MemorySpace.SMEM),
            pl.BlockSpec((embed.shape[0], D), lambda i: (0, 0)),
            pl.BlockSpec((BLK, D), lambda i: (i, 0)),
        ],
        out_specs=pl.BlockSpec((BLK, D), lambda i: (i, 0)),
        out_shape=jax.ShapeDtypeStruct((TC_ROWS, D), tokens.dtype),
        compiler_params=pltpu.CompilerParams(
            vmem_limit_bytes=60 * 1024 * 1024,
        ),
    )(idx, embed, tok2)

    sc = functools.partial(
        pl.kernel,
        mesh=plsc.VectorSubcoreMesh(core_axis_name="c", subcore_axis_name="s"),
        out_type=jax.ShapeDtypeStruct((SC_ROWS, D), jnp.float32),
        scratch_types=[
            pltpu.VMEM((8,), jnp.int32),
            pltpu.VMEM((8, D), jnp.float32),
            pltpu.VMEM((2, CH, D), jnp.float32),
            pltpu.SemaphoreType.DMA,
            pltpu.SemaphoreType.DMA,
            pltpu.SemaphoreType.DMA,
            pltpu.SemaphoreType.DMA,
        ],
    )(_sc_body)
    sc_out = sc(tok2, embed, idx8)

    out = jnp.concatenate([tc_out, sc_out], axis=0)
    return out.reshape(B, N, D)


# speedup vs baseline: 2.9223x; 2.9223x over previous
import jax
import jax.numpy as jnp
from jax.experimental import pallas as pl
from jax.experimental.pallas import tpu as pltpu

B, N, D = 32, 576, 768
ROWS = B * N
BLK = 4608


def _body(idx_ref, embed_ref, tok_ref, out_ref):
    row = embed_ref[pl.ds(idx_ref[0], 1), :]
    out_ref[...] = tok_ref[...] + row


def kernel(tokens, embed, frame_idx):
    idx = jnp.asarray(frame_idx, dtype=jnp.int32).reshape((1,))
    tok2 = tokens.reshape(ROWS, D)
    out = pl.pallas_call(
        _body,
        grid=(ROWS // BLK,),
        in_specs=[
            pl.BlockSpec(memory_space=pltpu.MemorySpace.SMEM),
            pl.BlockSpec((embed.shape[0], D), lambda i: (0, 0)),
            pl.BlockSpec((BLK, D), lambda i: (i, 0)),
        ],
        out_specs=pl.BlockSpec((BLK, D), lambda i: (i, 0)),
        out_shape=jax.ShapeDtypeStruct((ROWS, D), tokens.dtype),
        compiler_params=pltpu.CompilerParams(
            vmem_limit_bytes=60 * 1024 * 1024,
        ),
    )(idx, embed, tok2)
    return out.reshape(B, N, D)
